# baseline (speedup 1.0000x reference)
import functools

import jax
import jax.numpy as jnp
from jax import lax
from jax.experimental import pallas as pl
from jax.experimental.pallas import tpu as pltpu
from jax.experimental.pallas import tpu_sc as plsc

_B, _T, _E = 4, 8192, 1024
_NW = 32
_ROWS = _B * _T
_ROWS_PER_W = _ROWS // _NW
_RCHUNK = 32
_CELEMS = _RCHUNK * _E
_NCHUNK = _ROWS_PER_W // _RCHUNK
_UNROLL = 8
_LANES = 16


def _sc_body(x_hbm, pos_hbm, out_hbm, xbuf, pbuf):
    c = lax.axis_index("c")
    s = lax.axis_index("s")
    wid = s * 2 + c
    row0 = wid * _ROWS_PER_W
    trow0 = lax.rem(row0, _T)

    def chunk(k, carry):
        el = (row0 + k * _RCHUNK) * _E
        pel = (trow0 + k * _RCHUNK) * _E
        pltpu.sync_copy(x_hbm.at[pl.ds(el, _CELEMS)], xbuf)
        pltpu.sync_copy(pos_hbm.at[pl.ds(pel, _CELEMS)], pbuf)

        def add16(i, carry2):
            base = i * (_LANES * _UNROLL)
            for u in range(_UNROLL):
                off = base + u * _LANES
                xbuf[pl.ds(off, _LANES)] = (
                    xbuf[pl.ds(off, _LANES)] + pbuf[pl.ds(off, _LANES)]
                )
            return carry2

        lax.fori_loop(0, _CELEMS // (_LANES * _UNROLL), add16, 0)
        pltpu.sync_copy(xbuf, out_hbm.at[pl.ds(el, _CELEMS)])
        return carry

    lax.fori_loop(0, _NCHUNK, chunk, 0)


@jax.jit
def _sc_add(x_flat, pos_flat):
    mesh = plsc.VectorSubcoreMesh(core_axis_name="c", subcore_axis_name="s")
    return pl.kernel(
        _sc_body,
        mesh=mesh,
        out_type=jax.ShapeDtypeStruct((_B * _T * _E,), jnp.float32),
        scratch_types=[
            pltpu.VMEM((_CELEMS,), jnp.float32),
            pltpu.VMEM((_CELEMS,), jnp.float32),
        ],
    )(x_flat, pos_flat)


def kernel(x, pos_embedding):
    B, T, E = x.shape
    out = _sc_add(x.reshape(-1), pos_embedding.reshape(-1))
    return out.reshape(B, T, E)
